# baseline (speedup 1.0000x reference)
import math

import jax
import jax.numpy as jnp
import numpy as np
from jax.experimental import pallas as pl
from jax.experimental.pallas import tpu as pltpu

_N = 96
_P = _N * _N
_L = 2
_HIGH = jax.lax.Precision.HIGHEST


def _gelu(t):
    return t * 0.5 * (1.0 + jax.lax.erf(t * np.float32(1.0 / math.sqrt(2.0))))


def _dot(a, b, precision=_HIGH):
    return jax.lax.dot_general(a, b, (((1,), (0,)), ((), ())),
                               precision=precision,
                               preferred_element_type=jnp.float32)


def _mlp(w1_ref, b1_ref, w2_ref, b2_ref, act):
    h = _gelu(_dot(w1_ref[...], act, None) + b1_ref[...])
    return _dot(w2_ref[...], h, None) + b2_ref[...]


def _tree_sum_src(v):
    for w in (4608, 2304, 1152, 576, 288):
        v = v[:, :w] + v[:, w:]
    return v[:, :96] + v[:, 96:192] + v[:, 192:288]


def _round_bf16(v):
    return v.astype(jnp.bfloat16).astype(jnp.float32)


def _body(xT_ref, nw1t_ref, nb1_ref, nw2t_ref, nb2_ref,
          ew1t_ref, eb1_ref, ew2t_ref, eb2_ref,
          v2ew1t_ref, v2eb1_ref, v2ew2t_ref, v2eb2_ref,
          e2vw1t_ref, e2vb1_ref, e2vw2t_ref, e2vb2_ref,
          hw1t_ref, hb1_ref, hw2t_ref, hb2_ref, scale_ref,
          Rs_ref, efd_ref, out_ref):
    bf16 = jnp.bfloat16
    f32 = jnp.float32

    def _split(a):
        a_hi = a.astype(bf16)
        a_lo = (a - a_hi.astype(f32)).astype(bf16)
        return a_hi, a_lo

    Rs = Rs_ref[...]
    inv = jnp.float32(1.0 / (_N - 1))
    sp = jnp.log1p(jnp.exp(scale_ref[...]))
    for g in range(xT_ref.shape[0]):
        X = xT_ref[g]

        h_v = _mlp(nw1t_ref, nb1_ref, nw2t_ref, nb2_ref, X)

        X_hi, X_lo = _split(X)
        h_v_bf = h_v.astype(bf16)
        s0 = _dot(jnp.concatenate([X_hi, X_lo, h_v_bf], axis=0),
                  Rs, None)
        X_src = s0[0:2] + s0[2:4]
        h_v_src = s0[4:20]
        X_dst = jnp.concatenate([X] * _N, axis=1)
        dr = X_dst - X_src
        r2 = jnp.sum(dr * dr, axis=0, keepdims=True)
        rr = jnp.sqrt(r2 + 1e-12)
        e_feat = jnp.concatenate([dr, rr, r2], axis=0)
        h_e = _mlp(ew1t_ref, eb1_ref, ew2t_ref, eb2_ref, e_feat)
        h_e_d = _mlp(ew1t_ref, eb1_ref, ew2t_ref, eb2_ref, efd_ref[...])

        for l in range(_L):
            if l > 0:
                h_v_bf = h_v.astype(bf16)
                h_v_src = _dot(h_v_bf, Rs, None)
            cat = jnp.concatenate([h_v_src, h_e], axis=0)
            h_e = _mlp(v2ew1t_ref[l], v2eb1_ref[l],
                       v2ew2t_ref[l], v2eb2_ref[l], cat)
            u = _gelu(_dot(e2vw1t_ref[l], h_e, None) + e2vb1_ref[l])

            cat_d = jnp.concatenate([h_v_bf.astype(f32), h_e_d], axis=0)
            h_e_d = _mlp(v2ew1t_ref[l], v2eb1_ref[l],
                         v2ew2t_ref[l], v2eb2_ref[l], cat_d)
            u_d = _gelu(_dot(e2vw1t_ref[l], h_e_d, None) + e2vb1_ref[l])

            P = _tree_sum_src(_round_bf16(u)) - _round_bf16(u_d)
            P_hi, P_lo = _split(P)
            w2t = e2vw2t_ref[l]
            w2P = _dot(w2t, P_hi, None) + _dot(w2t, P_lo, None)
            h_v = h_v + w2P * inv + e2vb2_ref[l]

        t = jnp.tanh(_dot(hw1t_ref[...], h_v, None) + hb1_ref[...])
        dx = _dot(hw2t_ref[...], t, None) + hb2_ref[...]
        dx = dx * sp
        dx = dx - jnp.mean(dx, axis=1, keepdims=True)
        out_ref[g] = dx


def kernel(x, node_w1, node_b1, node_w2, node_b2, edge_w1, edge_b1, edge_w2,
           edge_b2, v2e_w1, v2e_b1, v2e_w2, v2e_b2, e2v_w1, e2v_b1, e2v_w2,
           e2v_b2, head_w1, head_b1, head_w2, head_b2, scale):
    B, N, D = x.shape
    omega = 1.0
    xT = jnp.transpose(x, (0, 2, 1)) * np.float32(math.sqrt(omega))

    eye = np.eye(_N, dtype=np.float32)
    Rs = np.kron(eye, np.ones((1, _N), dtype=np.float32))
    efd = np.zeros((D + 2, _N), dtype=np.float32)
    efd[D, :] = 1e-6

    f32 = jnp.float32
    args = (
        xT,
        node_w1[:D].T.astype(f32), node_b1[:, None],
        node_w2.T, node_b2[:, None],
        edge_w1.T, edge_b1[:, None],
        edge_w2.T, edge_b2[:, None],
        jnp.transpose(v2e_w1, (0, 2, 1)), v2e_b1[:, :, None],
        jnp.transpose(v2e_w2, (0, 2, 1)), v2e_b2[:, :, None],
        jnp.transpose(e2v_w1, (0, 2, 1)), e2v_b1[:, :, None],
        jnp.transpose(e2v_w2, (0, 2, 1)).astype(jnp.bfloat16),
        e2v_b2[:, :, None],
        head_w1.T, head_b1[:, None],
        head_w2.T, head_b2[:, None],
        jnp.reshape(scale, (1, 1)),
        jnp.asarray(Rs, dtype=jnp.bfloat16),
        jnp.asarray(efd),
    )

    def full(a):
        return pl.BlockSpec(a.shape, lambda b, _nd=a.ndim: (0,) * _nd)

    G = 16
    in_specs = [pl.BlockSpec((G, D, N), lambda b: (b, 0, 0))]
    in_specs += [full(a) for a in args[1:]]

    out = pl.pallas_call(
        _body,
        grid=(B // G,),
        in_specs=in_specs,
        out_specs=pl.BlockSpec((G, D, N), lambda b: (b, 0, 0)),
        out_shape=jax.ShapeDtypeStruct((B, D, N), jnp.float32),
        compiler_params=pltpu.CompilerParams(
            dimension_semantics=("parallel",),
        ),
    )(*args)
    return jnp.transpose(out, (0, 2, 1))


# speedup vs baseline: 1.1265x; 1.1265x over previous
import math

import jax
import jax.numpy as jnp
import numpy as np
from jax.experimental import pallas as pl
from jax.experimental.pallas import tpu as pltpu

_N = 96
_P = _N * _N
_L = 2
_HIGH = jax.lax.Precision.HIGHEST


def _gelu(t):
    return t * 0.5 * (1.0 + jax.lax.erf(t * np.float32(1.0 / math.sqrt(2.0))))


def _dot(a, b, precision=_HIGH):
    return jax.lax.dot_general(a, b, (((1,), (0,)), ((), ())),
                               precision=precision,
                               preferred_element_type=jnp.float32)


def _mlp(w1_ref, b1_ref, w2_ref, b2_ref, act):
    h = _gelu(_dot(w1_ref[...], act, None) + b1_ref[...])
    return _dot(w2_ref[...], h, None) + b2_ref[...]


def _tree_sum_src(v):
    for w in (4608, 2304, 1152, 576, 288):
        v = v[:, :w] + v[:, w:]
    return v[:, :96] + v[:, 96:192] + v[:, 192:288]


def _round_bf16(v):
    return v.astype(jnp.bfloat16).astype(jnp.float32)


def _body(xT_ref, nw1t_ref, nb1_ref, nw2t_ref, nb2_ref,
          ew1t_ref, eb1_ref, ew2t_ref, eb2_ref,
          v2ew1t_ref, v2eb1_ref, v2ew2t_ref, v2eb2_ref,
          e2vw1t_ref, e2vb1_ref, e2vw2t_ref, e2vb2_ref,
          hw1t_ref, hb1_ref, hw2t_ref, hb2_ref, scale_ref,
          Rs_ref, efd_ref, out_ref):
    bf16 = jnp.bfloat16
    f32 = jnp.float32

    def _split(a):
        a_hi = a.astype(bf16)
        a_lo = (a - a_hi.astype(f32)).astype(bf16)
        return a_hi, a_lo

    Rs = Rs_ref[...]
    inv = jnp.float32(1.0 / (_N - 1))
    sp = jnp.log1p(jnp.exp(scale_ref[...]))
    for g in range(xT_ref.shape[0]):
        X = xT_ref[g]

        h_v = _mlp(nw1t_ref, nb1_ref, nw2t_ref, nb2_ref, X)

        X_src = jnp.repeat(X, _N, axis=1)
        X_dst = jnp.concatenate([X] * _N, axis=1)
        dr = X_dst - X_src
        r2 = jnp.sum(dr * dr, axis=0, keepdims=True)
        rr = jnp.sqrt(r2 + 1e-12)
        e_feat = jnp.concatenate([dr, rr, r2], axis=0)
        h_e = _mlp(ew1t_ref, eb1_ref, ew2t_ref, eb2_ref, e_feat)
        h_e_d = _mlp(ew1t_ref, eb1_ref, ew2t_ref, eb2_ref, efd_ref[...])

        for l in range(_L):
            h_v_bf = h_v.astype(bf16)
            h_v_src = jnp.repeat(h_v_bf, _N, axis=1).astype(f32)
            cat = jnp.concatenate([h_v_src, h_e], axis=0)
            h_e = _mlp(v2ew1t_ref[l], v2eb1_ref[l],
                       v2ew2t_ref[l], v2eb2_ref[l], cat)
            u = _gelu(_dot(e2vw1t_ref[l], h_e, None) + e2vb1_ref[l])

            cat_d = jnp.concatenate([h_v_bf.astype(f32), h_e_d], axis=0)
            h_e_d = _mlp(v2ew1t_ref[l], v2eb1_ref[l],
                         v2ew2t_ref[l], v2eb2_ref[l], cat_d)
            u_d = _gelu(_dot(e2vw1t_ref[l], h_e_d, None) + e2vb1_ref[l])

            P = _tree_sum_src(_round_bf16(u)) - _round_bf16(u_d)
            P_hi, P_lo = _split(P)
            w2t = e2vw2t_ref[l]
            w2P = _dot(w2t, P_hi, None) + _dot(w2t, P_lo, None)
            h_v = h_v + w2P * inv + e2vb2_ref[l]

        t = jnp.tanh(_dot(hw1t_ref[...], h_v, None) + hb1_ref[...])
        dx = _dot(hw2t_ref[...], t, None) + hb2_ref[...]
        dx = dx * sp
        dx = dx - jnp.mean(dx, axis=1, keepdims=True)
        out_ref[g] = dx


def kernel(x, node_w1, node_b1, node_w2, node_b2, edge_w1, edge_b1, edge_w2,
           edge_b2, v2e_w1, v2e_b1, v2e_w2, v2e_b2, e2v_w1, e2v_b1, e2v_w2,
           e2v_b2, head_w1, head_b1, head_w2, head_b2, scale):
    B, N, D = x.shape
    omega = 1.0
    xT = jnp.transpose(x, (0, 2, 1)) * np.float32(math.sqrt(omega))

    eye = np.eye(_N, dtype=np.float32)
    Rs = np.kron(eye, np.ones((1, _N), dtype=np.float32))
    efd = np.zeros((D + 2, _N), dtype=np.float32)
    efd[D, :] = 1e-6

    f32 = jnp.float32
    args = (
        xT,
        node_w1[:D].T.astype(f32), node_b1[:, None],
        node_w2.T, node_b2[:, None],
        edge_w1.T, edge_b1[:, None],
        edge_w2.T, edge_b2[:, None],
        jnp.transpose(v2e_w1, (0, 2, 1)), v2e_b1[:, :, None],
        jnp.transpose(v2e_w2, (0, 2, 1)), v2e_b2[:, :, None],
        jnp.transpose(e2v_w1, (0, 2, 1)), e2v_b1[:, :, None],
        jnp.transpose(e2v_w2, (0, 2, 1)).astype(jnp.bfloat16),
        e2v_b2[:, :, None],
        head_w1.T, head_b1[:, None],
        head_w2.T, head_b2[:, None],
        jnp.reshape(scale, (1, 1)),
        jnp.asarray(Rs, dtype=jnp.bfloat16),
        jnp.asarray(efd),
    )

    def full(a):
        return pl.BlockSpec(a.shape, lambda b, _nd=a.ndim: (0,) * _nd)

    G = 8
    in_specs = [pl.BlockSpec((G, D, N), lambda b: (b, 0, 0))]
    in_specs += [full(a) for a in args[1:]]

    out = pl.pallas_call(
        _body,
        grid=(B // G,),
        in_specs=in_specs,
        out_specs=pl.BlockSpec((G, D, N), lambda b: (b, 0, 0)),
        out_shape=jax.ShapeDtypeStruct((B, D, N), jnp.float32),
        compiler_params=pltpu.CompilerParams(
            dimension_semantics=("parallel",),
        ),
    )(*args)
    return jnp.transpose(out, (0, 2, 1))


# speedup vs baseline: 1.1802x; 1.0477x over previous
import math

import jax
import jax.numpy as jnp
import numpy as np
from jax.experimental import pallas as pl
from jax.experimental.pallas import tpu as pltpu

_N = 96
_P = _N * _N
_L = 2
_HIGH = jax.lax.Precision.HIGHEST


def _gelu(t):
    return t * 0.5 * (1.0 + jax.lax.erf(t * np.float32(1.0 / math.sqrt(2.0))))


def _dot(a, b, precision=_HIGH):
    return jax.lax.dot_general(a, b, (((1,), (0,)), ((), ())),
                               precision=precision,
                               preferred_element_type=jnp.float32)


def _mlp(w1_ref, b1_ref, w2_ref, b2_ref, act):
    h = _gelu(_dot(w1_ref[...], act, None) + b1_ref[...])
    return _dot(w2_ref[...], h, None) + b2_ref[...]


def _tree_sum_src(v):
    for w in (4608, 2304, 1152, 576, 288):
        v = v[:, :w] + v[:, w:]
    return v[:, :96] + v[:, 96:192] + v[:, 192:288]


def _round_bf16(v):
    return v.astype(jnp.bfloat16).astype(jnp.float32)


def _body(xT_ref, nw1t_ref, nb1_ref, nw2t_ref, nb2_ref,
          ew1t_ref, eb1_ref, ew2t_ref, eb2_ref,
          v2ew1t_ref, v2eb1_ref, v2ew2t_ref, v2eb2_ref,
          e2vw1t_ref, e2vb1_ref, e2vw2t_ref, e2vb2_ref,
          hw1t_ref, hb1_ref, hw2t_ref, hb2_ref, scale_ref,
          efd_ref, out_ref):
    bf16 = jnp.bfloat16
    f32 = jnp.float32

    def _split(a):
        a_hi = a.astype(bf16)
        a_lo = (a - a_hi.astype(f32)).astype(bf16)
        return a_hi, a_lo

    inv = jnp.float32(1.0 / (_N - 1))
    sp = jnp.log1p(jnp.exp(scale_ref[...]))
    for g in range(xT_ref.shape[0]):
        X = xT_ref[g]

        h_v = _mlp(nw1t_ref, nb1_ref, nw2t_ref, nb2_ref, X)

        X_src = jnp.repeat(X, _N, axis=1)
        X_dst = jnp.concatenate([X] * _N, axis=1)
        dr = X_dst - X_src
        r2 = jnp.sum(dr * dr, axis=0, keepdims=True)
        rr = jnp.sqrt(r2 + 1e-12)
        e_feat = jnp.concatenate([dr, rr, r2], axis=0)
        h_e = _mlp(ew1t_ref, eb1_ref, ew2t_ref, eb2_ref, e_feat)
        h_e_d = _mlp(ew1t_ref, eb1_ref, ew2t_ref, eb2_ref, efd_ref[...])

        for l in range(_L):
            h_v_bf = h_v.astype(bf16)
            h_v_src = jnp.repeat(h_v_bf, _N, axis=1).astype(f32)
            cat = jnp.concatenate([h_v_src, h_e], axis=0)
            h_e = _mlp(v2ew1t_ref[l], v2eb1_ref[l],
                       v2ew2t_ref[l], v2eb2_ref[l], cat)
            u = _gelu(_dot(e2vw1t_ref[l], h_e, None) + e2vb1_ref[l])

            cat_d = jnp.concatenate([h_v_bf.astype(f32), h_e_d], axis=0)
            h_e_d = _mlp(v2ew1t_ref[l], v2eb1_ref[l],
                         v2ew2t_ref[l], v2eb2_ref[l], cat_d)
            u_d = _gelu(_dot(e2vw1t_ref[l], h_e_d, None) + e2vb1_ref[l])

            P = _tree_sum_src(_round_bf16(u)) - _round_bf16(u_d)
            P_hi, P_lo = _split(P)
            w2t = e2vw2t_ref[l]
            w2P = _dot(w2t, P_hi, None) + _dot(w2t, P_lo, None)
            h_v = h_v + w2P * inv + e2vb2_ref[l]

        t = jnp.tanh(_dot(hw1t_ref[...], h_v, None) + hb1_ref[...])
        dx = _dot(hw2t_ref[...], t, None) + hb2_ref[...]
        dx = dx * sp
        dx = dx - jnp.mean(dx, axis=1, keepdims=True)
        out_ref[g] = dx


def kernel(x, node_w1, node_b1, node_w2, node_b2, edge_w1, edge_b1, edge_w2,
           edge_b2, v2e_w1, v2e_b1, v2e_w2, v2e_b2, e2v_w1, e2v_b1, e2v_w2,
           e2v_b2, head_w1, head_b1, head_w2, head_b2, scale):
    B, N, D = x.shape
    omega = 1.0
    xT = jnp.transpose(x, (0, 2, 1)) * np.float32(math.sqrt(omega))

    efd = np.zeros((D + 2, _N), dtype=np.float32)
    efd[D, :] = 1e-6

    f32 = jnp.float32
    args = (
        xT,
        node_w1[:D].T.astype(f32), node_b1[:, None],
        node_w2.T, node_b2[:, None],
        edge_w1.T, edge_b1[:, None],
        edge_w2.T, edge_b2[:, None],
        jnp.transpose(v2e_w1, (0, 2, 1)), v2e_b1[:, :, None],
        jnp.transpose(v2e_w2, (0, 2, 1)), v2e_b2[:, :, None],
        jnp.transpose(e2v_w1, (0, 2, 1)), e2v_b1[:, :, None],
        jnp.transpose(e2v_w2, (0, 2, 1)).astype(jnp.bfloat16),
        e2v_b2[:, :, None],
        head_w1.T, head_b1[:, None],
        head_w2.T, head_b2[:, None],
        jnp.reshape(scale, (1, 1)),
        jnp.asarray(efd),
    )

    def full(a):
        return pl.BlockSpec(a.shape, lambda b, _nd=a.ndim: (0,) * _nd)

    G = 8
    in_specs = [pl.BlockSpec((G, D, N), lambda b: (b, 0, 0))]
    in_specs += [full(a) for a in args[1:]]

    out = pl.pallas_call(
        _body,
        grid=(B // G,),
        in_specs=in_specs,
        out_specs=pl.BlockSpec((G, D, N), lambda b: (b, 0, 0)),
        out_shape=jax.ShapeDtypeStruct((B, D, N), jnp.float32),
        compiler_params=pltpu.CompilerParams(
            dimension_semantics=("parallel",),
        ),
    )(*args)
    return jnp.transpose(out, (0, 2, 1))
